# baseline (speedup 1.0000x reference)
import functools

import jax
import jax.numpy as jnp
from jax.experimental import pallas as pl
from jax.experimental.pallas import tpu as pltpu


def _body(t_ref, x_ref, w1_ref, b1_ref, w2_ref, b2_ref, wp_ref, gc_ref,
          o_ref, *, n_types, out_d, hidden, n_split):
    full = x_ref.shape[0]
    t_all = t_ref[...]
    lanes = jax.lax.broadcasted_iota(jnp.int32, (full, 128), 1)
    onehot = (lanes == t_all).astype(jnp.bfloat16)
    corr = jnp.dot(onehot, gc_ref[...],
                   preferred_element_type=jnp.float32)

    hm = full // n_split
    for s in range(n_split):
        rows = slice(s * hm, (s + 1) * hm)
        x = x_ref[rows, :].astype(jnp.bfloat16)
        h = jnp.dot(x, w1_ref[...], preferred_element_type=jnp.float32)
        h = jnp.maximum(h + b1_ref[...], 0.0)
        h = jnp.dot(h.astype(jnp.bfloat16), w2_ref[...],
                    preferred_element_type=jnp.float32)
        h = h + b2_ref[...]
        y = jnp.dot(h.astype(jnp.bfloat16), wp_ref[...],
                    preferred_element_type=jnp.float32)
        inv_h = 1.0 / hidden
        mu = jnp.sum(h, axis=-1, keepdims=True) * inv_h
        m2 = jnp.sum(h * h, axis=-1, keepdims=True) * inv_h
        rstd = jax.lax.rsqrt(jnp.maximum(m2 - mu * mu, 0.0) + 1e-5)

        t = t_all[rows, :]
        g_sel = corr[rows, :out_d]
        c_sel = corr[rows, out_d:]
        y_sel = y[:, (n_types - 1) * out_d:]
        for tt in range(n_types - 2, -1, -1):
            y_sel = jnp.where(t == tt, y[:, tt * out_d:(tt + 1) * out_d],
                              y_sel)
        o_ref[rows, :] = rstd * y_sel - (rstd * mu) * g_sel + c_sel


def kernel(node_latent, node_types, w1, b1, w2, b2, ln_gamma, ln_beta,
           head_w, head_b, *, interpret=False, bm=1000, n_split=2):
    n, latent = node_latent.shape
    hidden = w1.shape[1]
    out_d = head_w.shape[2]
    n_types = head_w.shape[0]
    grid = (n // bm,)

    t2 = node_types.reshape(n, 1)
    b1r = b1.reshape(1, hidden)
    b2r = b2.reshape(1, hidden)
    w1b = w1.astype(jnp.bfloat16)
    w2b = w2.astype(jnp.bfloat16)
    w_cat = head_w.transpose(1, 0, 2).reshape(hidden, n_types * out_d)
    wp = (ln_gamma[:, None] * w_cat).astype(jnp.bfloat16)
    g1 = (ln_gamma @ w_cat).reshape(n_types, out_d)
    c_all = (ln_beta @ w_cat).reshape(n_types, out_d) + head_b
    gc = jnp.zeros((128, 2 * out_d), jnp.float32)
    gc = gc.at[:n_types, :out_d].set(g1).at[:n_types, out_d:].set(c_all)
    gcb = gc.astype(jnp.bfloat16)

    return pl.pallas_call(
        functools.partial(_body, n_types=n_types, out_d=out_d, hidden=hidden,
                          n_split=n_split),
        out_shape=jax.ShapeDtypeStruct((n, out_d), jnp.float32),
        grid=grid,
        in_specs=[
            pl.BlockSpec((bm, 1), lambda i: (i, 0)),
            pl.BlockSpec((bm, latent), lambda i: (i, 0)),
            pl.BlockSpec((latent, hidden), lambda i: (0, 0)),
            pl.BlockSpec((1, hidden), lambda i: (0, 0)),
            pl.BlockSpec((hidden, hidden), lambda i: (0, 0)),
            pl.BlockSpec((1, hidden), lambda i: (0, 0)),
            pl.BlockSpec((hidden, n_types * out_d), lambda i: (0, 0)),
            pl.BlockSpec((128, 2 * out_d), lambda i: (0, 0)),
        ],
        out_specs=pl.BlockSpec((bm, out_d), lambda i: (i, 0)),
        compiler_params=pltpu.CompilerParams(
            dimension_semantics=("parallel",),
            vmem_limit_bytes=56 * 1024 * 1024,
        ),
        name="adaptive_decoder",
        interpret=interpret,
    )(t2, node_latent, w1b, b1r, w2b, b2r, wp, gcb)


# speedup vs baseline: 1.0498x; 1.0498x over previous
import functools

import jax
import jax.numpy as jnp
from jax.experimental import pallas as pl
from jax.experimental.pallas import tpu as pltpu

_BM = 1000


def _body(t_ref, x_ref, w1_ref, b1_ref, w2_ref, b2_ref, wp_ref, gc_ref,
          o_ref, *, n_types, out_d, hidden):
    x = x_ref[...].astype(jnp.bfloat16)
    h = jnp.dot(x, w1_ref[...], preferred_element_type=jnp.float32)
    h = jnp.maximum(h + b1_ref[...], 0.0)
    h = jnp.dot(h.astype(jnp.bfloat16), w2_ref[...],
                preferred_element_type=jnp.float32)
    h = h + b2_ref[...]
    y = jnp.dot(h.astype(jnp.bfloat16), wp_ref[...],
                preferred_element_type=jnp.float32)
    inv_h = 1.0 / hidden
    mu = jnp.sum(h, axis=-1, keepdims=True) * inv_h
    m2 = jnp.sum(h * h, axis=-1, keepdims=True) * inv_h
    rstd = jax.lax.rsqrt(jnp.maximum(m2 - mu * mu, 0.0) + 1e-5)

    t = t_ref[...]
    bm = t.shape[0]
    lanes = jax.lax.broadcasted_iota(jnp.int32, (bm, 128), 1)
    onehot = (lanes == t).astype(jnp.bfloat16)
    corr = jnp.dot(onehot, gc_ref[...],
                   preferred_element_type=jnp.float32)
    g_sel = corr[:, :out_d]
    c_sel = corr[:, out_d:]

    y_sel = y[:, (n_types - 1) * out_d:]
    for tt in range(n_types - 2, -1, -1):
        y_sel = jnp.where(t == tt, y[:, tt * out_d:(tt + 1) * out_d], y_sel)
    o_ref[...] = rstd * y_sel - (rstd * mu) * g_sel + c_sel


def kernel(node_latent, node_types, w1, b1, w2, b2, ln_gamma, ln_beta,
           head_w, head_b):
    n, latent = node_latent.shape
    hidden = w1.shape[1]
    out_d = head_w.shape[2]
    n_types = head_w.shape[0]
    grid = (n // _BM,)

    t2 = node_types.reshape(n, 1)
    b1r = b1.reshape(1, hidden)
    b2r = b2.reshape(1, hidden)
    w1b = w1.astype(jnp.bfloat16)
    w2b = w2.astype(jnp.bfloat16)
    w_cat = head_w.transpose(1, 0, 2).reshape(hidden, n_types * out_d)
    wp = (ln_gamma[:, None] * w_cat).astype(jnp.bfloat16)
    g1 = (ln_gamma @ w_cat).reshape(n_types, out_d)
    c_all = (ln_beta @ w_cat).reshape(n_types, out_d) + head_b
    gc = jnp.zeros((128, 2 * out_d), jnp.float32)
    gc = gc.at[:n_types, :out_d].set(g1).at[:n_types, out_d:].set(c_all)
    gcb = gc.astype(jnp.bfloat16)

    return pl.pallas_call(
        functools.partial(_body, n_types=n_types, out_d=out_d, hidden=hidden),
        out_shape=jax.ShapeDtypeStruct((n, out_d), jnp.float32),
        grid=grid,
        in_specs=[
            pl.BlockSpec((_BM, 1), lambda i: (i, 0)),
            pl.BlockSpec((_BM, latent), lambda i: (i, 0)),
            pl.BlockSpec((latent, hidden), lambda i: (0, 0)),
            pl.BlockSpec((1, hidden), lambda i: (0, 0)),
            pl.BlockSpec((hidden, hidden), lambda i: (0, 0)),
            pl.BlockSpec((1, hidden), lambda i: (0, 0)),
            pl.BlockSpec((hidden, n_types * out_d), lambda i: (0, 0)),
            pl.BlockSpec((128, 2 * out_d), lambda i: (0, 0)),
        ],
        out_specs=pl.BlockSpec((_BM, out_d), lambda i: (i, 0)),
        compiler_params=pltpu.CompilerParams(
            dimension_semantics=("parallel",),
            vmem_limit_bytes=56 * 1024 * 1024,
        ),
        name="adaptive_decoder",
    )(t2, node_latent, w1b, b1r, w2b, b2r, wp, gcb)


# speedup vs baseline: 1.0739x; 1.0229x over previous
import functools

import jax
import jax.numpy as jnp
from jax.experimental import pallas as pl
from jax.experimental.pallas import tpu as pltpu

_BM = 1000


def _prep_body(w1_ref, w2_ref, hw_ref, gb_ref, hb_ref,
               w1b_ref, w2b_ref, wp_ref, gcb_ref, *, n_types, out_d):
    w1b_ref[...] = w1_ref[...].astype(jnp.bfloat16)
    w2b_ref[...] = w2_ref[...].astype(jnp.bfloat16)
    gcol = gb_ref[:, 0:1]
    bcol = gb_ref[:, 1:2]
    gcb_ref[...] = jnp.zeros(gcb_ref.shape, jnp.bfloat16)
    for t in range(n_types):
        hw = hw_ref[t]
        gw = gcol * hw
        wp_ref[:, t * out_d:(t + 1) * out_d] = gw.astype(jnp.bfloat16)
        g1t = jnp.sum(gw, axis=0, keepdims=True)
        ct = jnp.sum(bcol * hw, axis=0, keepdims=True) + hb_ref[t][None, :]
        gcb_ref[t:t + 1, 0:out_d] = g1t.astype(jnp.bfloat16)
        gcb_ref[t:t + 1, out_d:2 * out_d] = ct.astype(jnp.bfloat16)


def _body(t_ref, x_ref, w1_ref, b1_ref, w2_ref, b2_ref, wp_ref, gc_ref,
          o_ref, *, n_types, out_d, hidden):
    x = x_ref[...].astype(jnp.bfloat16)
    h = jnp.dot(x, w1_ref[...], preferred_element_type=jnp.float32)
    h = jnp.maximum(h + b1_ref[...], 0.0)
    h = jnp.dot(h.astype(jnp.bfloat16), w2_ref[...],
                preferred_element_type=jnp.float32)
    h = h + b2_ref[...]
    y = jnp.dot(h.astype(jnp.bfloat16), wp_ref[...],
                preferred_element_type=jnp.float32)
    inv_h = 1.0 / hidden
    mu = jnp.sum(h, axis=-1, keepdims=True) * inv_h
    m2 = jnp.sum(h * h, axis=-1, keepdims=True) * inv_h
    rstd = jax.lax.rsqrt(jnp.maximum(m2 - mu * mu, 0.0) + 1e-5)

    t = t_ref[...]
    bm = t.shape[0]
    lanes = jax.lax.broadcasted_iota(jnp.int32, (bm, 128), 1)
    onehot = (lanes == t).astype(jnp.bfloat16)
    corr = jnp.dot(onehot, gc_ref[...],
                   preferred_element_type=jnp.float32)
    g_sel = corr[:, :out_d]
    c_sel = corr[:, out_d:]

    y_sel = y[:, (n_types - 1) * out_d:]
    for tt in range(n_types - 2, -1, -1):
        y_sel = jnp.where(t == tt, y[:, tt * out_d:(tt + 1) * out_d], y_sel)
    o_ref[...] = rstd * y_sel - (rstd * mu) * g_sel + c_sel


def kernel(node_latent, node_types, w1, b1, w2, b2, ln_gamma, ln_beta,
           head_w, head_b):
    n, latent = node_latent.shape
    hidden = w1.shape[1]
    out_d = head_w.shape[2]
    n_types = head_w.shape[0]
    grid = (n // _BM,)

    t2 = node_types.reshape(n, 1)
    b1r = b1.reshape(1, hidden)
    b2r = b2.reshape(1, hidden)
    gb = jnp.stack([ln_gamma, ln_beta], axis=1)

    w1b, w2b, wp, gcb = pl.pallas_call(
        functools.partial(_prep_body, n_types=n_types, out_d=out_d),
        out_shape=(
            jax.ShapeDtypeStruct((latent, hidden), jnp.bfloat16),
            jax.ShapeDtypeStruct((hidden, hidden), jnp.bfloat16),
            jax.ShapeDtypeStruct((hidden, n_types * out_d), jnp.bfloat16),
            jax.ShapeDtypeStruct((128, 2 * out_d), jnp.bfloat16),
        ),
        name="decoder_weight_prep",
    )(w1, w2, head_w, gb, head_b)

    return pl.pallas_call(
        functools.partial(_body, n_types=n_types, out_d=out_d, hidden=hidden),
        out_shape=jax.ShapeDtypeStruct((n, out_d), jnp.float32),
        grid=grid,
        in_specs=[
            pl.BlockSpec((_BM, 1), lambda i: (i, 0)),
            pl.BlockSpec((_BM, latent), lambda i: (i, 0)),
            pl.BlockSpec((latent, hidden), lambda i: (0, 0)),
            pl.BlockSpec((1, hidden), lambda i: (0, 0)),
            pl.BlockSpec((hidden, hidden), lambda i: (0, 0)),
            pl.BlockSpec((1, hidden), lambda i: (0, 0)),
            pl.BlockSpec((hidden, n_types * out_d), lambda i: (0, 0)),
            pl.BlockSpec((128, 2 * out_d), lambda i: (0, 0)),
        ],
        out_specs=pl.BlockSpec((_BM, out_d), lambda i: (i, 0)),
        compiler_params=pltpu.CompilerParams(
            dimension_semantics=("parallel",),
            vmem_limit_bytes=56 * 1024 * 1024,
        ),
        name="adaptive_decoder",
    )(t2, node_latent, w1b, b1r, w2b, b2r, wp, gcb)
